# baseline (speedup 1.0000x reference)
import functools

import jax
import jax.numpy as jnp
import numpy as np
from jax import lax
from jax.experimental import pallas as pl
from jax.experimental.pallas import tpu as pltpu
from jax.experimental.pallas import tpu_sc as plsc

_NC = 2
_NS = 16
_CH = 80
_K = 5
_R = 640

_MASK_HI = np.uint32(0xFFFF0000)
_HALF = np.uint32(0x00008000)


def _sc_gather(table, idx, c, Bc):
    C = table.shape[1]
    nw = _NC * _NS
    b_per_w = Bc // nw
    n_ch = b_per_w // _CH
    mesh = plsc.VectorSubcoreMesh(core_axis_name="c", subcore_axis_name="s")

    @functools.partial(
        pl.kernel,
        mesh=mesh,
        out_type=jax.ShapeDtypeStruct((Bc, C), jnp.float32),
        scratch_types=[
            pltpu.VMEM((b_per_w,), jnp.int32),
            pltpu.VMEM((2, _CH, C), jnp.float32),
            pltpu.SemaphoreType.DMA,
            pltpu.SemaphoreType.DMA,
        ],
        name=f"sc_gather_c{c}",
    )
    def gather_k(table_hbm, idx_hbm, out_hbm, idx_v, rows_v, gsem, wsem):
        wid = lax.axis_index("s") * _NC + lax.axis_index("c")
        base = wid * b_per_w
        pltpu.sync_copy(idx_hbm.at[pl.ds(c * Bc + base, b_per_w)], idx_v)

        def g_start(k, b):
            pltpu.async_copy(
                table_hbm.at[idx_v.at[pl.ds(k * _CH, _CH)]], rows_v.at[b], gsem)

        def g_wait(b):
            pltpu.make_async_copy(
                table_hbm.at[pl.ds(0, _CH)], rows_v.at[b], gsem).wait()

        def w_start(k, b):
            pltpu.async_copy(
                rows_v.at[b], out_hbm.at[pl.ds(base + k * _CH, _CH)], wsem)

        def w_wait(b):
            pltpu.make_async_copy(
                rows_v.at[b], out_hbm.at[pl.ds(0, _CH)], wsem).wait()

        g_start(0, 0)

        def body(j, carry):
            for b in (0, 1):
                k = 2 * j + b
                g_wait(b)

                @pl.when(k >= 1)
                def _():
                    w_wait(1 - b)

                @pl.when(k + 1 < n_ch)
                def _():
                    g_start(k + 1, 1 - b)

                w_start(k, b)
            return carry

        lax.fori_loop(0, n_ch // 2, body, 0)
        w_wait(1)

    return gather_k(table, idx)


def _tc_matmul_stats(x, g, wt, c, y_prev):
    E, C = x.shape
    H = C // 2
    Tc = g.shape[0] // (4 * _R)

    def body(x_ref, n0_ref, n1_ref, n2_ref, n3_ref, wt_ref, *rest):
        y_ref, st_ref = rest[-2], rest[-1]
        n0 = n0_ref[...]
        n1 = n1_ref[...]
        n2 = n2_ref[...]
        n3 = n3_ref[...]
        feat = jnp.concatenate(
            [x_ref[...],
             jnp.minimum(n0, n1), jnp.maximum(n0, n1),
             jnp.minimum(n2, n3), jnp.maximum(n2, n3)], axis=1)
        y = jnp.dot(feat, wt_ref[...], preferred_element_type=jnp.float32)

        @pl.when(pl.program_id(0) == 0)
        def _():
            st_ref[...] = jnp.zeros_like(st_ref)

        st_ref[...] += jnp.stack([jnp.sum(y, axis=0), jnp.sum(y * y, axis=0)])

        u = lax.bitcast_convert_type(y, jnp.uint32)
        hi = (u[:, :H] + _HALF) & _MASK_HI
        lo = (u[:, H:] + _HALF) >> 16
        y_ref[...] = lax.bitcast_convert_type(hi | lo, jnp.float32)

    in_specs = [
        pl.BlockSpec((_R, C), lambda i: (c * Tc + i, 0)),
        pl.BlockSpec((_R, C), lambda i, _j=0: (_j * Tc + i, 0)),
        pl.BlockSpec((_R, C), lambda i, _j=1: (_j * Tc + i, 0)),
        pl.BlockSpec((_R, C), lambda i, _j=2: (_j * Tc + i, 0)),
        pl.BlockSpec((_R, C), lambda i, _j=3: (_j * Tc + i, 0)),
        pl.BlockSpec((5 * C, C), lambda i: (0, 0)),
    ]
    operands = [x, g, g, g, g, wt]
    aliases = {}
    if y_prev is not None:
        in_specs.append(pl.BlockSpec(memory_space=pltpu.MemorySpace.HBM))
        operands.append(y_prev)
        aliases = {6: 0}

    return pl.pallas_call(
        body,
        grid=(Tc,),
        in_specs=in_specs,
        out_specs=[
            pl.BlockSpec((_R, H), lambda i: (c * Tc + i, 0)),
            pl.BlockSpec((2, C), lambda i: (0, 0)),
        ],
        out_shape=[
            jax.ShapeDtypeStruct((E, H), jnp.float32),
            jax.ShapeDtypeStruct((2, C), jnp.float32),
        ],
        input_output_aliases=aliases,
    )(*operands)


def _tc_norm(y, st, gamma, beta):
    E, H = y.shape
    C = 2 * H
    T = E // _R
    inv_e = 1.0 / E

    def body(y_ref, st_ref, gm_ref, bt_ref, o_ref):
        mean = st_ref[0:1, :] * inv_e
        var = st_ref[1:2, :] * inv_e - mean * mean
        scale = gm_ref[...] * lax.rsqrt(var + 1e-5)
        shift = bt_ref[...] - mean * scale
        u = lax.bitcast_convert_type(y_ref[...], jnp.uint32)
        yhi = lax.bitcast_convert_type(u & _MASK_HI, jnp.float32)
        ylo = lax.bitcast_convert_type(u << 16, jnp.float32)
        ohi = jnp.maximum(yhi * scale[:, :H] + shift[:, :H], 0.0)
        olo = jnp.maximum(ylo * scale[:, H:] + shift[:, H:], 0.0)
        o_ref[...] = jnp.concatenate([ohi, olo], axis=1)

    return pl.pallas_call(
        body,
        grid=(T,),
        in_specs=[
            pl.BlockSpec((_R, H), lambda i: (i, 0)),
            pl.BlockSpec((2, C), lambda i: (0, 0)),
            pl.BlockSpec((1, C), lambda i: (0, 0)),
            pl.BlockSpec((1, C), lambda i: (0, 0)),
        ],
        out_specs=pl.BlockSpec((_R, C), lambda i: (i, 0)),
        out_shape=jax.ShapeDtypeStruct((E, C), jnp.float32),
    )(y, st, gamma.reshape(1, C), beta.reshape(1, C))


def kernel(x, nb, W, gamma, beta):
    E, C = x.shape
    Ec = E // _K
    Bc = 4 * Ec
    idx = (jnp.clip(nb, 0, E - 1).astype(jnp.int32)
           .reshape(_K, Ec, 4).transpose(0, 2, 1).reshape(-1))
    wt = W.T
    gs = [_sc_gather(x, idx, c, Bc) for c in range(_K)]
    y = None
    sts = []
    for c in range(_K):
        y, st_c = _tc_matmul_stats(x, gs[c], wt, c, y)
        sts.append(st_c)
    st = sts[0]
    for st_c in sts[1:]:
        st = st + st_c
    return _tc_norm(y, st, gamma, beta)


# speedup vs baseline: 1.0565x; 1.0565x over previous
import functools

import jax
import jax.numpy as jnp
import numpy as np
from jax import lax
from jax.experimental import pallas as pl
from jax.experimental.pallas import tpu as pltpu
from jax.experimental.pallas import tpu_sc as plsc

_NC = 2
_NS = 16
_CH = 80
_K = 5
_R = 640

_MASK_HI = np.uint32(0xFFFF0000)
_HALF = np.uint32(0x00008000)


def _sc_gather(table, idx, c, Bc):
    C = table.shape[1]
    nw = _NC * _NS
    b_per_w = Bc // nw
    n_ch = b_per_w // _CH
    mesh = plsc.VectorSubcoreMesh(core_axis_name="c", subcore_axis_name="s")

    @functools.partial(
        pl.kernel,
        mesh=mesh,
        out_type=jax.ShapeDtypeStruct((Bc, C), jnp.float32),
        scratch_types=[
            pltpu.VMEM((b_per_w,), jnp.int32),
            pltpu.VMEM((4, _CH, C), jnp.float32),
            pltpu.SemaphoreType.DMA,
            pltpu.SemaphoreType.DMA,
            pltpu.SemaphoreType.DMA,
            pltpu.SemaphoreType.DMA,
        ],
        name=f"sc_gather_c{c}",
    )
    def gather_k(table_hbm, idx_hbm, out_hbm, idx_v, rows_v,
                 gsem0, gsem1, wsem0, wsem1):
        wid = lax.axis_index("s") * _NC + lax.axis_index("c")
        base = wid * b_per_w
        pltpu.sync_copy(idx_hbm.at[pl.ds(c * Bc + base, b_per_w)], idx_v)
        gsems = (gsem0, gsem1)
        wsems = (wsem0, wsem1)

        def g_start(k, b):
            pltpu.async_copy(
                table_hbm.at[idx_v.at[pl.ds(k * _CH, _CH)]], rows_v.at[b],
                gsems[b % 2])

        def g_wait(b):
            pltpu.make_async_copy(
                table_hbm.at[pl.ds(0, _CH)], rows_v.at[b], gsems[b % 2]).wait()

        def w_start(k, b):
            pltpu.async_copy(
                rows_v.at[b], out_hbm.at[pl.ds(base + k * _CH, _CH)],
                wsems[b % 2])

        def w_wait(b):
            pltpu.make_async_copy(
                rows_v.at[b], out_hbm.at[pl.ds(0, _CH)], wsems[b % 2]).wait()

        def step(k, b, bn, tail):
            g_wait(b)

            @pl.when(k >= 2)
            def _():
                w_wait(bn)

            if not tail:
                @pl.when(k + 2 < n_ch)
                def _():
                    g_start(k + 2, bn)

            w_start(k, b)

        g_start(0, 0)
        g_start(1, 1)
        n_main = (n_ch // 4) * 4

        def body(j, carry):
            for t in range(4):
                k = 4 * j + t
                step(k, t, (t + 2) % 4, False)
            return carry

        lax.fori_loop(0, n_main // 4, body, 0)
        for k in range(n_main, n_ch):
            step(k, k % 4, (k + 2) % 4, k + 2 >= n_ch)
        w_wait((n_ch - 2) % 4)
        w_wait((n_ch - 1) % 4)

    return gather_k(table, idx)


def _tc_matmul_stats(x, g, wt, c, y_prev):
    E, C = x.shape
    H = C // 2
    Tc = g.shape[0] // (4 * _R)

    def body(x_ref, n0_ref, n1_ref, n2_ref, n3_ref, wt_ref, *rest):
        y_ref, st_ref = rest[-2], rest[-1]
        n0 = n0_ref[...]
        n1 = n1_ref[...]
        n2 = n2_ref[...]
        n3 = n3_ref[...]
        feat = jnp.concatenate(
            [x_ref[...],
             jnp.minimum(n0, n1), jnp.maximum(n0, n1),
             jnp.minimum(n2, n3), jnp.maximum(n2, n3)], axis=1)
        y = jnp.dot(feat, wt_ref[...], preferred_element_type=jnp.float32)

        @pl.when(pl.program_id(0) == 0)
        def _():
            st_ref[...] = jnp.zeros_like(st_ref)

        st_ref[...] += jnp.stack([jnp.sum(y, axis=0), jnp.sum(y * y, axis=0)])

        u = lax.bitcast_convert_type(y, jnp.uint32)
        hi = (u[:, :H] + _HALF) & _MASK_HI
        lo = (u[:, H:] + _HALF) >> 16
        y_ref[...] = lax.bitcast_convert_type(hi | lo, jnp.float32)

    in_specs = [
        pl.BlockSpec((_R, C), lambda i: (c * Tc + i, 0)),
        pl.BlockSpec((_R, C), lambda i, _j=0: (_j * Tc + i, 0)),
        pl.BlockSpec((_R, C), lambda i, _j=1: (_j * Tc + i, 0)),
        pl.BlockSpec((_R, C), lambda i, _j=2: (_j * Tc + i, 0)),
        pl.BlockSpec((_R, C), lambda i, _j=3: (_j * Tc + i, 0)),
        pl.BlockSpec((5 * C, C), lambda i: (0, 0)),
    ]
    operands = [x, g, g, g, g, wt]
    aliases = {}
    if y_prev is not None:
        in_specs.append(pl.BlockSpec(memory_space=pltpu.MemorySpace.HBM))
        operands.append(y_prev)
        aliases = {6: 0}

    return pl.pallas_call(
        body,
        grid=(Tc,),
        in_specs=in_specs,
        out_specs=[
            pl.BlockSpec((_R, H), lambda i: (c * Tc + i, 0)),
            pl.BlockSpec((2, C), lambda i: (0, 0)),
        ],
        out_shape=[
            jax.ShapeDtypeStruct((E, H), jnp.float32),
            jax.ShapeDtypeStruct((2, C), jnp.float32),
        ],
        input_output_aliases=aliases,
    )(*operands)


def _tc_norm(y, st, gamma, beta):
    E, H = y.shape
    C = 2 * H
    T = E // _R
    inv_e = 1.0 / E

    def body(y_ref, st_ref, gm_ref, bt_ref, o_ref):
        mean = st_ref[0:1, :] * inv_e
        var = st_ref[1:2, :] * inv_e - mean * mean
        scale = gm_ref[...] * lax.rsqrt(var + 1e-5)
        shift = bt_ref[...] - mean * scale
        u = lax.bitcast_convert_type(y_ref[...], jnp.uint32)
        yhi = lax.bitcast_convert_type(u & _MASK_HI, jnp.float32)
        ylo = lax.bitcast_convert_type(u << 16, jnp.float32)
        ohi = jnp.maximum(yhi * scale[:, :H] + shift[:, :H], 0.0)
        olo = jnp.maximum(ylo * scale[:, H:] + shift[:, H:], 0.0)
        o_ref[...] = jnp.concatenate([ohi, olo], axis=1)

    return pl.pallas_call(
        body,
        grid=(T,),
        in_specs=[
            pl.BlockSpec((_R, H), lambda i: (i, 0)),
            pl.BlockSpec((2, C), lambda i: (0, 0)),
            pl.BlockSpec((1, C), lambda i: (0, 0)),
            pl.BlockSpec((1, C), lambda i: (0, 0)),
        ],
        out_specs=pl.BlockSpec((_R, C), lambda i: (i, 0)),
        out_shape=jax.ShapeDtypeStruct((E, C), jnp.float32),
    )(y, st, gamma.reshape(1, C), beta.reshape(1, C))


def kernel(x, nb, W, gamma, beta):
    E, C = x.shape
    Ec = E // _K
    Bc = 4 * Ec
    idx = (jnp.clip(nb, 0, E - 1).astype(jnp.int32)
           .reshape(_K, Ec, 4).transpose(0, 2, 1).reshape(-1))
    wt = W.T
    gs = [_sc_gather(x, idx, c, Bc) for c in range(_K)]
    y = None
    sts = []
    for c in range(_K):
        y, st_c = _tc_matmul_stats(x, gs[c], wt, c, y)
        sts.append(st_c)
    st = sts[0]
    for st_c in sts[1:]:
        st = st + st_c
    return _tc_norm(y, st, gamma, beta)


# speedup vs baseline: 1.0657x; 1.0088x over previous
import functools

import jax
import jax.numpy as jnp
import numpy as np
from jax import lax
from jax.experimental import pallas as pl
from jax.experimental.pallas import tpu as pltpu
from jax.experimental.pallas import tpu_sc as plsc

_NC = 2
_NS = 16
_CH = 80
_K = 10
_R = 640

_MASK_HI = np.uint32(0xFFFF0000)
_HALF = np.uint32(0x00008000)


def _sc_gather(table, idx, c, Bc):
    C = table.shape[1]
    nw = _NC * _NS
    b_per_w = Bc // nw
    n_ch = b_per_w // _CH
    mesh = plsc.VectorSubcoreMesh(core_axis_name="c", subcore_axis_name="s")

    @functools.partial(
        pl.kernel,
        mesh=mesh,
        out_type=jax.ShapeDtypeStruct((Bc, C), jnp.float32),
        scratch_types=[
            pltpu.VMEM((b_per_w,), jnp.int32),
            pltpu.VMEM((4, _CH, C), jnp.float32),
            pltpu.SemaphoreType.DMA,
            pltpu.SemaphoreType.DMA,
            pltpu.SemaphoreType.DMA,
            pltpu.SemaphoreType.DMA,
        ],
        name=f"sc_gather_c{c}",
    )
    def gather_k(table_hbm, idx_hbm, out_hbm, idx_v, rows_v,
                 gsem0, gsem1, wsem0, wsem1):
        wid = lax.axis_index("s") * _NC + lax.axis_index("c")
        base = wid * b_per_w
        pltpu.sync_copy(idx_hbm.at[pl.ds(c * Bc + base, b_per_w)], idx_v)
        gsems = (gsem0, gsem1)
        wsems = (wsem0, wsem1)

        def g_start(k, b):
            pltpu.async_copy(
                table_hbm.at[idx_v.at[pl.ds(k * _CH, _CH)]], rows_v.at[b],
                gsems[b % 2])

        def g_wait(b):
            pltpu.make_async_copy(
                table_hbm.at[pl.ds(0, _CH)], rows_v.at[b], gsems[b % 2]).wait()

        def w_start(k, b):
            pltpu.async_copy(
                rows_v.at[b], out_hbm.at[pl.ds(base + k * _CH, _CH)],
                wsems[b % 2])

        def w_wait(b):
            pltpu.make_async_copy(
                rows_v.at[b], out_hbm.at[pl.ds(0, _CH)], wsems[b % 2]).wait()

        def step(k, b, bn, tail):
            g_wait(b)

            @pl.when(k >= 2)
            def _():
                w_wait(bn)

            if not tail:
                @pl.when(k + 2 < n_ch)
                def _():
                    g_start(k + 2, bn)

            w_start(k, b)

        g_start(0, 0)
        g_start(1, 1)
        n_main = (n_ch // 4) * 4

        def body(j, carry):
            for t in range(4):
                k = 4 * j + t
                step(k, t, (t + 2) % 4, False)
            return carry

        lax.fori_loop(0, n_main // 4, body, 0)
        for k in range(n_main, n_ch):
            step(k, k % 4, (k + 2) % 4, k + 2 >= n_ch)
        w_wait((n_ch - 2) % 4)
        w_wait((n_ch - 1) % 4)

    return gather_k(table, idx)


def _tc_matmul_stats(x, g, wt, c, y_prev):
    E, C = x.shape
    H = C // 2
    Tc = g.shape[0] // (4 * _R)

    def body(x_ref, n0_ref, n1_ref, n2_ref, n3_ref, wt_ref, *rest):
        y_ref, st_ref = rest[-2], rest[-1]
        n0 = n0_ref[...]
        n1 = n1_ref[...]
        n2 = n2_ref[...]
        n3 = n3_ref[...]
        feat = jnp.concatenate(
            [x_ref[...],
             jnp.minimum(n0, n1), jnp.maximum(n0, n1),
             jnp.minimum(n2, n3), jnp.maximum(n2, n3)], axis=1)
        y = jnp.dot(feat, wt_ref[...], preferred_element_type=jnp.float32)

        @pl.when(pl.program_id(0) == 0)
        def _():
            st_ref[...] = jnp.zeros_like(st_ref)

        st_ref[...] += jnp.stack([jnp.sum(y, axis=0), jnp.sum(y * y, axis=0)])

        u = lax.bitcast_convert_type(y, jnp.uint32)
        hi = (u[:, :H] + _HALF) & _MASK_HI
        lo = (u[:, H:] + _HALF) >> 16
        y_ref[...] = lax.bitcast_convert_type(hi | lo, jnp.float32)

    in_specs = [
        pl.BlockSpec((_R, C), lambda i: (c * Tc + i, 0)),
        pl.BlockSpec((_R, C), lambda i, _j=0: (_j * Tc + i, 0)),
        pl.BlockSpec((_R, C), lambda i, _j=1: (_j * Tc + i, 0)),
        pl.BlockSpec((_R, C), lambda i, _j=2: (_j * Tc + i, 0)),
        pl.BlockSpec((_R, C), lambda i, _j=3: (_j * Tc + i, 0)),
        pl.BlockSpec((5 * C, C), lambda i: (0, 0)),
    ]
    operands = [x, g, g, g, g, wt]
    aliases = {}
    if y_prev is not None:
        in_specs.append(pl.BlockSpec(memory_space=pltpu.MemorySpace.HBM))
        operands.append(y_prev)
        aliases = {6: 0}

    return pl.pallas_call(
        body,
        grid=(Tc,),
        in_specs=in_specs,
        out_specs=[
            pl.BlockSpec((_R, H), lambda i: (c * Tc + i, 0)),
            pl.BlockSpec((2, C), lambda i: (0, 0)),
        ],
        out_shape=[
            jax.ShapeDtypeStruct((E, H), jnp.float32),
            jax.ShapeDtypeStruct((2, C), jnp.float32),
        ],
        input_output_aliases=aliases,
    )(*operands)


def _tc_norm(y, st, gamma, beta):
    E, H = y.shape
    C = 2 * H
    T = E // _R
    inv_e = 1.0 / E

    def body(y_ref, st_ref, gm_ref, bt_ref, o_ref):
        mean = st_ref[0:1, :] * inv_e
        var = st_ref[1:2, :] * inv_e - mean * mean
        scale = gm_ref[...] * lax.rsqrt(var + 1e-5)
        shift = bt_ref[...] - mean * scale
        u = lax.bitcast_convert_type(y_ref[...], jnp.uint32)
        yhi = lax.bitcast_convert_type(u & _MASK_HI, jnp.float32)
        ylo = lax.bitcast_convert_type(u << 16, jnp.float32)
        ohi = jnp.maximum(yhi * scale[:, :H] + shift[:, :H], 0.0)
        olo = jnp.maximum(ylo * scale[:, H:] + shift[:, H:], 0.0)
        o_ref[...] = jnp.concatenate([ohi, olo], axis=1)

    return pl.pallas_call(
        body,
        grid=(T,),
        in_specs=[
            pl.BlockSpec((_R, H), lambda i: (i, 0)),
            pl.BlockSpec((2, C), lambda i: (0, 0)),
            pl.BlockSpec((1, C), lambda i: (0, 0)),
            pl.BlockSpec((1, C), lambda i: (0, 0)),
        ],
        out_specs=pl.BlockSpec((_R, C), lambda i: (i, 0)),
        out_shape=jax.ShapeDtypeStruct((E, C), jnp.float32),
    )(y, st, gamma.reshape(1, C), beta.reshape(1, C))


def kernel(x, nb, W, gamma, beta):
    E, C = x.shape
    Ec = E // _K
    Bc = 4 * Ec
    idx = (nb.astype(jnp.int32)
           .reshape(_K, Ec, 4).transpose(0, 2, 1).reshape(-1))
    wt = W.T
    gs = [_sc_gather(x, idx, c, Bc) for c in range(_K)]
    y = None
    sts = []
    for c in range(_K):
        y, st_c = _tc_matmul_stats(x, gs[c], wt, c, y)
        sts.append(st_c)
    st = sts[0]
    for st_c in sts[1:]:
        st = st + st_c
    return _tc_norm(y, st, gamma, beta)
